# baseline (speedup 1.0000x reference)
import functools

import jax
import jax.numpy as jnp
from jax import lax
from jax.experimental import pallas as pl
from jax.experimental.pallas import tpu as pltpu
from jax.experimental.pallas import tpu_sc as plsc

E = 8
N_GROUP = 4
EPG = E // N_GROUP
ROUTED_SCALE = 2.5
Z_COEF = 0.001
BAL_COEF = 0.001

BT = 256
BD = 256
NBR = 40
NBE = 64
NW = 32


def _argmax_first(s, iota, big):
    m = jnp.max(s, axis=-1, keepdims=True)
    i = jnp.min(jnp.where(s == m, iota, big), axis=-1, keepdims=True)
    return m, i


def _router_body(nt_total, p_slots, x_ref, wr_ref, bias_ref,
                 tw_ref, posc_ref, be_ref, aux_ref,
                 idx_scr, w_scr, bc_scr, wsum_ref, cnt_ref, z_ref):
    ph = pl.program_id(0)
    nt = pl.program_id(1)
    tsl = pl.ds(nt * BT, BT)

    @pl.when(ph == 0)
    def _phase0():
        xb = x_ref[...]
        logits = lax.dot_general(xb, wr_ref[...], (((1,), (1,)), ((), ())),
                                 preferred_element_type=jnp.float32)
        scores = jax.nn.sigmoid(logits)
        sfc = scores + bias_ref[...]

        iota_e = lax.broadcasted_iota(jnp.int32, (BT, E), 1)
        iota_g = lax.broadcasted_iota(jnp.int32, (BT, N_GROUP), 1)

        gmap = (lax.broadcasted_iota(jnp.int32, (E, N_GROUP), 0) // EPG ==
                lax.broadcasted_iota(jnp.int32, (E, N_GROUP), 1)
                ).astype(jnp.float32)
        gs = jnp.dot(sfc, gmap, preferred_element_type=jnp.float32,
                     precision=lax.Precision.HIGHEST)

        _, g1 = _argmax_first(gs, iota_g, N_GROUP)
        gs2 = jnp.where(iota_g == g1, -jnp.inf, gs)
        _, g2 = _argmax_first(gs2, iota_g, N_GROUP)

        eg = iota_e // EPG
        sm = jnp.where((eg == g1) | (eg == g2), sfc, -1.0)
        _, e1 = _argmax_first(sm, iota_e, E)
        sm2 = jnp.where(iota_e == e1, -jnp.inf, sm)
        _, e2 = _argmax_first(sm2, iota_e, E)

        h1 = (iota_e == e1).astype(jnp.float32)
        h2 = (iota_e == e2).astype(jnp.float32)
        w1 = jnp.sum(h1 * scores, axis=-1, keepdims=True)
        w2 = jnp.sum(h2 * scores, axis=-1, keepdims=True)
        denom = w1 + w2 + 1e-20
        w1 = w1 / denom * ROUTED_SCALE
        w2 = w2 / denom * ROUTED_SCALE

        h = h1 + h2
        ltri = (lax.broadcasted_iota(jnp.int32, (BT, BT), 0) >
                lax.broadcasted_iota(jnp.int32, (BT, BT), 1)
                ).astype(jnp.float32)
        csum = jnp.dot(ltri, h, preferred_element_type=jnp.float32)
        lr1 = jnp.sum(h1 * csum, axis=-1, keepdims=True)
        lr2 = jnp.sum(h2 * csum, axis=-1, keepdims=True)

        idx_scr[tsl, :] = jnp.concatenate(
            [e1, e2, lr1.astype(jnp.int32), lr2.astype(jnp.int32),
             jnp.zeros((BT, 4), jnp.int32)], axis=1)
        w_scr[tsl, :] = jnp.concatenate([w1, w2], axis=1)
        bc = jnp.sum(h, axis=0, keepdims=True)
        bc_scr[pl.ds(nt, 1), :] = bc

        @pl.when(nt == 0)
        def _():
            wsum_ref[...] = jnp.zeros_like(wsum_ref)
            cnt_ref[...] = jnp.zeros_like(cnt_ref)
            z_ref[0] = 0.0

        wsum_ref[...] += jnp.sum(h1 * w1 + h2 * w2, axis=0, keepdims=True)
        cnt_ref[...] += bc
        mx = jnp.max(logits, axis=-1, keepdims=True)
        lse = jnp.log(jnp.sum(jnp.exp(logits - mx), axis=-1,
                              keepdims=True)) + mx
        z_ref[0] += jnp.sum(lse * lse)

        @pl.when(nt == nt_total - 1)
        def _():
            t_tot = jnp.float32(nt_total * BT)
            mean_load = wsum_ref[...] / t_tot
            freq = cnt_ref[...] / t_tot
            balance = E * jnp.sum(mean_load * freq)
            aux_ref[...] = jnp.broadcast_to(
                BAL_COEF * balance + Z_COEF * (z_ref[0] / t_tot), (1, 1))

    @pl.when(ph == 1)
    def _phase1():
        bcs = bc_scr[...]
        tot = jnp.sum(bcs, axis=0, keepdims=True)
        padded = jnp.floor((tot + (BD - 1)) / BD) * BD
        umap = (lax.broadcasted_iota(jnp.int32, (E, E), 0) <
                lax.broadcasted_iota(jnp.int32, (E, E), 1)
                ).astype(jnp.float32)
        offs = jnp.dot(padded, umap, preferred_element_type=jnp.float32,
                       precision=lax.Precision.HIGHEST)
        iota_nt = lax.broadcasted_iota(jnp.int32, (bcs.shape[0], E), 0)
        runb = jnp.sum(jnp.where(iota_nt < nt, bcs, 0.0), axis=0,
                       keepdims=True)
        base = offs + runb

        idx = idx_scr[tsl, :]
        iota_e = lax.broadcasted_iota(jnp.int32, (BT, E), 1)
        h1 = (iota_e == idx[:, 0:1]).astype(jnp.float32)
        h2 = (iota_e == idx[:, 1:2]).astype(jnp.float32)
        baseb = jnp.broadcast_to(base, (BT, E))
        p1 = jnp.sum(h1 * baseb, axis=-1, keepdims=True) + \
            idx[:, 2:3].astype(jnp.float32)
        p2 = jnp.sum(h2 * baseb, axis=-1, keepdims=True) + \
            idx[:, 3:4].astype(jnp.float32)
        posc_ref[...] = jnp.concatenate(
            [p1.astype(jnp.int32), p2.astype(jnp.int32)], axis=1)
        tw_ref[...] = w_scr[tsl, :]

        @pl.when(nt == 0)
        def _():
            cum = offs + padded
            cumb = jnp.broadcast_to(cum, (NBE, E))
            jstart = (lax.broadcasted_iota(jnp.int32, (NBE, E), 0) *
                      BD).astype(jnp.float32)
            be = jnp.sum((jstart >= cumb).astype(jnp.int32), axis=-1,
                         keepdims=True)
            be_ref[...] = jnp.minimum(be, E - 1)


def _ffn(xb, gw, uw, dw):
    g = lax.dot_general(xb, gw, (((1,), (1,)), ((), ())),
                        preferred_element_type=jnp.float32)
    u = lax.dot_general(xb, uw, (((1,), (1,)), ((), ())),
                        preferred_element_type=jnp.float32)
    h = g * jax.nn.sigmoid(g) * u
    return lax.dot_general(h, dw, (((1,), (1,)), ((), ())),
                           preferred_element_type=jnp.float32)


def _ffn_routed_body(be_ref, x_ref, gw_ref, uw_ref, dw_ref, out_ref):
    out_ref[...] = _ffn(x_ref[...], gw_ref[0], uw_ref[0], dw_ref[0])


def _ffn_shared_body(x_ref, gw_ref, uw_ref, dw_ref, out_ref):
    out_ref[...] = _ffn(x_ref[...], gw_ref[...], uw_ref[...], dw_ref[...])


def _combine_body(yg_ref, ysh_ref, tw_ref, out_ref):
    w1 = tw_ref[:, 0:1]
    w2 = tw_ref[:, 1:2]
    out_ref[...] = w1 * yg_ref[:, 0, :] + w2 * yg_ref[:, 1, :] + ysh_ref[...]


def _sc_wid():
    return lax.axis_index("s") * 2 + lax.axis_index("c")


def _scatter_body(apw, slots, posc_hbm, zeros_hbm, part_hbm, posv, tfs):
    wid = _sc_wid()
    nrow = apw // 16
    pltpu.sync_copy(zeros_hbm, tfs)
    pltpu.sync_copy(posc_hbm.at[pl.ds(wid * nrow, nrow)], posv)
    half = lax.shift_right_logical(lax.iota(jnp.int32, 16), 1)
    wbase = jnp.full((16,), wid * (apw // 2), jnp.int32)
    for j in range(nrow):
        v = posv[j, :]
        tok = wbase + (half + (j * 8))
        plsc.store_scatter(tfs, [v], tok)
    pltpu.sync_copy(tfs, part_hbm.at[pl.ds(wid * slots, slots)])


def _merge_gather_body(spw, slots, part_hbm, x_hbm, disp_hbm,
                       pbuf, tfs2, rows, sem):
    wid = _sc_wid()
    for p in range(NW):
        pltpu.sync_copy(part_hbm.at[pl.ds(p * slots + wid * spw, spw)],
                        pbuf.at[pl.ds(p * spw, spw)])
    for j in range(spw // 16):
        acc = pbuf[pl.ds(16 * j, 16)]
        for p in range(1, NW):
            acc = acc + pbuf[pl.ds(p * spw + 16 * j, 16)]
        tfs2[pl.ds(16 * j, 16)] = acc
    for c in range(spw // 64):
        pltpu.async_copy(x_hbm.at[tfs2.at[pl.ds(c * 64, 64)]],
                         rows, sem).wait()
        pltpu.sync_copy(rows, disp_hbm.at[pl.ds(wid * spw + c * 64, 64)])


def _combine_gather_body(apw, posc_hbm, y_hbm, yg_hbm, idxb, rows, sem):
    wid = _sc_wid()
    pltpu.sync_copy(posc_hbm.at[pl.ds(wid * apw, apw)], idxb)
    for c in range(apw // 64):
        pltpu.async_copy(y_hbm.at[idxb.at[pl.ds(c * 64, 64)]],
                         rows, sem).wait()
        pltpu.sync_copy(rows, yg_hbm.at[pl.ds(wid * apw + c * 64, 64)])


def kernel(hidden_states, router_weight, e_score_correction_bias, gate_w,
           up_w, down_w, shared_gate_w, shared_up_w, shared_down_w):
    bsz, seq, d = hidden_states.shape
    t = bsz * seq
    i_dim = gate_w.shape[1]
    nt = t // BT
    t2 = 2 * t
    p_slots = NBR * BD
    x = hidden_states.reshape(t, d)
    bias2 = e_score_correction_bias.reshape(1, E)

    tw, posc, be, aux = pl.pallas_call(
        functools.partial(_router_body, nt, p_slots),
        grid=(2, nt),
        in_specs=[
            pl.BlockSpec((BT, d), lambda ph, n: (n * (1 - ph), 0)),
            pl.BlockSpec((E, d), lambda ph, n: (0, 0)),
            pl.BlockSpec((1, E), lambda ph, n: (0, 0)),
        ],
        out_specs=[
            pl.BlockSpec((BT, 2), lambda ph, n: (n, 0)),
            pl.BlockSpec((BT, 2), lambda ph, n: (n, 0)),
            pl.BlockSpec((NBE, 1), lambda ph, n: (0, 0)),
            pl.BlockSpec((1, 1), lambda ph, n: (0, 0)),
        ],
        out_shape=[
            jax.ShapeDtypeStruct((t, 2), jnp.float32),
            jax.ShapeDtypeStruct((t, 2), jnp.int32),
            jax.ShapeDtypeStruct((NBE, 1), jnp.int32),
            jax.ShapeDtypeStruct((1, 1), jnp.float32),
        ],
        scratch_shapes=[
            pltpu.VMEM((t, 8), jnp.int32),
            pltpu.VMEM((t, 2), jnp.float32),
            pltpu.VMEM((nt, E), jnp.float32),
            pltpu.VMEM((1, E), jnp.float32),
            pltpu.VMEM((1, E), jnp.float32),
            pltpu.SMEM((1,), jnp.float32),
        ],
    )(x, router_weight, bias2)

    mesh = plsc.VectorSubcoreMesh(core_axis_name="c", subcore_axis_name="s")
    apw = t2 // NW
    spw = p_slots // NW
    posc16 = posc.reshape(t2 // 16, 16)
    zeros_p = jnp.zeros((p_slots,), jnp.int32)

    part = pl.kernel(
        functools.partial(_scatter_body, apw, p_slots),
        mesh=mesh,
        compiler_params=pltpu.CompilerParams(needs_layout_passes=False),
        out_type=jax.ShapeDtypeStruct((NW * p_slots,), jnp.int32),
        scratch_types=[
            pltpu.VMEM((apw // 16, 16), jnp.int32),
            pltpu.VMEM((p_slots,), jnp.int32),
        ],
    )(posc16, zeros_p)

    disp = pl.kernel(
        functools.partial(_merge_gather_body, spw, p_slots),
        mesh=mesh,
        compiler_params=pltpu.CompilerParams(needs_layout_passes=False),
        out_type=jax.ShapeDtypeStruct((p_slots, d), jnp.float32),
        scratch_types=[
            pltpu.VMEM((NW * spw,), jnp.int32),
            pltpu.VMEM((spw,), jnp.int32),
            pltpu.VMEM((64, d), jnp.float32),
            pltpu.SemaphoreType.DMA,
        ],
    )(part, x)

    y = pl.pallas_call(
        _ffn_routed_body,
        grid_spec=pltpu.PrefetchScalarGridSpec(
            num_scalar_prefetch=1,
            grid=(NBR,),
            in_specs=[
                pl.BlockSpec((BD, d), lambda b, be_r: (b, 0)),
                pl.BlockSpec((1, i_dim, d), lambda b, be_r: (be_r[b], 0, 0)),
                pl.BlockSpec((1, i_dim, d), lambda b, be_r: (be_r[b], 0, 0)),
                pl.BlockSpec((1, d, i_dim), lambda b, be_r: (be_r[b], 0, 0)),
            ],
            out_specs=pl.BlockSpec((BD, d), lambda b, be_r: (b, 0)),
        ),
        out_shape=jax.ShapeDtypeStruct((p_slots, d), jnp.float32),
    )(be.reshape(NBE), disp, gate_w, up_w, down_w)

    ysh = pl.pallas_call(
        _ffn_shared_body,
        grid=(nt,),
        in_specs=[
            pl.BlockSpec((BT, d), lambda n: (n, 0)),
            pl.BlockSpec((i_dim, d), lambda n: (0, 0)),
            pl.BlockSpec((i_dim, d), lambda n: (0, 0)),
            pl.BlockSpec((d, i_dim), lambda n: (0, 0)),
        ],
        out_specs=pl.BlockSpec((BT, d), lambda n: (n, 0)),
        out_shape=jax.ShapeDtypeStruct((t, d), jnp.float32),
    )(x, shared_gate_w, shared_up_w, shared_down_w)

    yg = pl.kernel(
        functools.partial(_combine_gather_body, apw),
        mesh=mesh,
        compiler_params=pltpu.CompilerParams(needs_layout_passes=False),
        out_type=jax.ShapeDtypeStruct((t2, d), jnp.float32),
        scratch_types=[
            pltpu.VMEM((apw,), jnp.int32),
            pltpu.VMEM((64, d), jnp.float32),
            pltpu.SemaphoreType.DMA,
        ],
    )(posc.reshape(t2), y)

    out = pl.pallas_call(
        _combine_body,
        grid=(nt,),
        in_specs=[
            pl.BlockSpec((BT, 2, d), lambda n: (n, 0, 0)),
            pl.BlockSpec((BT, d), lambda n: (n, 0)),
            pl.BlockSpec((BT, 2), lambda n: (n, 0)),
        ],
        out_specs=pl.BlockSpec((BT, d), lambda n: (n, 0)),
        out_shape=jax.ShapeDtypeStruct((t, d), jnp.float32),
    )(yg.reshape(t, 2, d), ysh, tw)

    return out.reshape(bsz, seq, d), aux[0, 0]
